# baseline (speedup 1.0000x reference)
import functools

import jax
import jax.numpy as jnp
from jax import lax
from jax.experimental import pallas as pl
from jax.experimental.pallas import tpu as pltpu
from jax.experimental.pallas import tpu_sc as plsc

N = 10000
NPAD = 10240
E = 320000
D = 128
NG = 8
NCOUT = 2
NW = 32
EPW = E // NW
K = 80
NCHUNK = EPW // K
RPT = NPAD // 16

_mesh = plsc.VectorSubcoreMesh(core_axis_name="c", subcore_axis_name="s")

_GATHER_DNUMS = lax.GatherDimensionNumbers(
    offset_dims=(), collapsed_slice_dims=(0,), start_index_map=(0,))


def _lane_bcast(vec, l):
    idx = jnp.full((16, 1), l, jnp.int32)
    return lax.gather(vec, idx, _GATHER_DNUMS, (1,),
                      mode=lax.GatherScatterMode.PROMISE_IN_BOUNDS)


@functools.partial(
    pl.kernel,
    out_type=jax.ShapeDtypeStruct((2, NPAD), jnp.float32),
    mesh=_mesh,
    scratch_types=[
        pltpu.VMEM((K,), jnp.int32),
        pltpu.VMEM((K,), jnp.float32),
        pltpu.VMEM((RPT,), jnp.float32),
        pltpu.VMEM_SHARED((NPAD,), jnp.float32),
    ],
)
def _sc_deg(dst_hbm, w_hbm, out_hbm, dst_v, w_v, zb_v, acc_sh):
    c = lax.axis_index("c")
    s = lax.axis_index("s")
    wid = s * 2 + c
    for i in range(RPT // 16):
        zb_v[pl.ds(i * 16, 16)] = jnp.zeros((16,), jnp.float32)
    pltpu.sync_copy(zb_v, acc_sh.at[pl.ds(s * RPT, RPT)])
    plsc.subcore_barrier()

    def chunk(ci, carry):
        off = pl.multiple_of(wid * EPW + ci * K, 8)
        pltpu.sync_copy(dst_hbm.at[pl.ds(off, K)], dst_v)
        pltpu.sync_copy(w_hbm.at[pl.ds(off, K)], w_v)

        def absgrp(g, cc):
            w_v[pl.ds(g * 16, 16)] = jnp.abs(w_v[pl.ds(g * 16, 16)])
            return cc

        lax.fori_loop(0, K // 16, absgrp, 0)
        pltpu.sync_copy(w_v, acc_sh.at[dst_v], add=True)
        return carry

    lax.fori_loop(0, NCHUNK, chunk, 0)
    plsc.subcore_barrier()
    pltpu.sync_copy(acc_sh.at[pl.ds(s * RPT, RPT)],
                    out_hbm.at[c, pl.ds(s * RPT, RPT)])


KE = 160
EPWP = 10240
NCH2 = EPWP // KE
PADE = EPWP - EPW
_RBYTES = KE * D * 4


def _scale_rows(wbuf, rbuf):
    def grp(g, cc):
        wvec = jnp.abs(wbuf[pl.ds(g * 16, 16)])
        base = g * 16
        for l in range(16):
            sv = _lane_bcast(wvec, l)
            e = base + l
            for j in range(D // 16):
                rbuf[e, pl.ds(j * 16, 16)] = rbuf[e, pl.ds(j * 16, 16)] * sv
        return cc

    lax.fori_loop(0, KE // 16, grp, 0)


@functools.partial(
    pl.kernel,
    out_type=jax.ShapeDtypeStruct((2, NPAD, D), jnp.float32),
    mesh=_mesh,
    scratch_types=[
        pltpu.VMEM((2, KE), jnp.int32),
        pltpu.VMEM((2, KE), jnp.int32),
        pltpu.VMEM((KE,), jnp.int32),
        pltpu.VMEM((KE,), jnp.int32),
        pltpu.VMEM((KE,), jnp.int32),
        pltpu.VMEM((KE,), jnp.int32),
        pltpu.VMEM((KE,), jnp.float32),
        pltpu.VMEM((KE,), jnp.float32),
        pltpu.VMEM((KE, D), jnp.float32),
        pltpu.VMEM((KE, D), jnp.float32),
        pltpu.VMEM_SHARED((NPAD, D), jnp.float32),
        pltpu.SemaphoreType.DMA,
        pltpu.SemaphoreType.DMA,
        pltpu.SemaphoreType.DMA,
        pltpu.SemaphoreType.DMA,
    ],
)
def _sc_edge(hp_hbm, epk_hbm, ew_hbm, out_hbm,
             ebuf0, ebuf1, srcb0, srcb1, dstb0, dstb1, wbuf0, wbuf1,
             rbuf0, rbuf1, acc_sh, gsem0, gsem1, ssem0, ssem1):
    c = lax.axis_index("c")
    s = lax.axis_index("s")
    wid = s * 2 + c
    ebuf = (ebuf0, ebuf1)
    srcb = (srcb0, srcb1)
    dstb = (dstb0, dstb1)
    wbuf = (wbuf0, wbuf1)
    rbuf = (rbuf0, rbuf1)
    gsem = (gsem0, gsem1)
    ssem = (ssem0, ssem1)
    row0 = wid * NCH2

    pltpu.sync_copy(hp_hbm.at[pl.ds(s * RPT, RPT)], acc_sh.at[pl.ds(s * RPT, RPT)])

    def zrow(r, cc):
        for j in range(D // 16):
            rbuf1[r, pl.ds(j * 16, 16)] = jnp.zeros((16,), jnp.float32)
        return cc

    lax.fori_loop(0, KE, zrow, 0)
    for g in range(KE // 16):
        dstb1[pl.ds(g * 16, 16)] = jnp.zeros((16,), jnp.int32)
    plsc.subcore_barrier()
    pltpu.async_copy(rbuf1, acc_sh.at[dstb1], ssem1, add=True)

    def _load_edata(j, o):
        pltpu.sync_copy(epk_hbm.at[row0 + j], ebuf[o])
        pltpu.sync_copy(ew_hbm.at[row0 + j], wbuf[o])

        def dcp(g, cc):
            srcb[o][pl.ds(g * 16, 16)] = ebuf[o][0, pl.ds(g * 16, 16)]
            dstb[o][pl.ds(g * 16, 16)] = ebuf[o][1, pl.ds(g * 16, 16)]
            return cc

        lax.fori_loop(0, KE // 16, dcp, 0)

    _load_edata(0, 0)
    pltpu.async_copy(hp_hbm.at[srcb0], rbuf0, gsem0)

    def pair(t, carry):
        for b in (0, 1):
            i = 2 * t + b
            o = 1 - b
            j = i + 1

            @pl.when(j < NCH2)
            def _():
                pltpu.make_async_copy(hp_hbm.at[pl.ds(0, KE)], rbuf[o],
                                      ssem[o]).wait()
                _load_edata(j, o)
                pltpu.async_copy(hp_hbm.at[srcb[o]], rbuf[o], gsem[o])

            pltpu.make_async_copy(hp_hbm.at[pl.ds(0, KE)], rbuf[b],
                                  gsem[b]).wait()
            _scale_rows(wbuf[b], rbuf[b])
            pltpu.async_copy(rbuf[b], acc_sh.at[dstb[b]], ssem[b],
                             add=True)
        return carry

    lax.fori_loop(0, NCH2 // 2, pair, 0)
    for b in (0, 1):
        pltpu.make_async_copy(hp_hbm.at[pl.ds(0, KE)], rbuf[b], ssem[b]).wait()
    plsc.subcore_barrier()
    pltpu.sync_copy(acc_sh.at[pl.ds(s * RPT, RPT)],
                    out_hbm.at[c, pl.ds(s * RPT, RPT)])


def _t_dinv_body(dp_ref, out_ref):
    dp = dp_ref[...]
    out_ref[...] = lax.rsqrt(1.0 + dp[0] + dp[1])


def _t_dinv(deg_parts):
    return pl.pallas_call(
        _t_dinv_body,
        out_shape=jax.ShapeDtypeStruct((NPAD,), jnp.float32),
    )(deg_parts)


def _store_h_hp(h, dv, h_ref, hp_ref):
    zpad = jnp.zeros((NPAD - N, D), jnp.float32)
    h_ref[0:N, :] = h
    h_ref[N:NPAD, :] = zpad
    hp_ref[0:N, :] = dv * h
    hp_ref[N:NPAD, :] = zpad


def _t_mm0_body(x_ref, w_ref, dv_ref, h_ref, hp_ref):
    h = jnp.dot(x_ref[...], w_ref[...], preferred_element_type=jnp.float32)
    _store_h_hp(h, dv_ref[0:N, :], h_ref, hp_ref)


_H_OUT = [jax.ShapeDtypeStruct((NPAD, D), jnp.float32),
          jax.ShapeDtypeStruct((NPAD, D), jnp.float32)]


def _t_mm0(x, W, dinv_col):
    return pl.pallas_call(_t_mm0_body, out_shape=_H_OUT)(x, W, dinv_col)


def _bn_block(parts_ref, h_ref, dv_ref, b_ref, g_ref, be_ref):
    p = parts_ref[0, 0:N, :] + parts_ref[1, 0:N, :]
    dv = dv_ref[0:N, :]
    h = h_ref[0:N, :]
    hp = dv * h
    z = dv * (p - hp) + b_ref[...]
    z = jnp.where(z > 0, z, 0.2 * z)
    m = jnp.sum(z, axis=0) * (1.0 / N)
    zc = z - m
    v = jnp.sum(zc * zc, axis=0) * (1.0 / N)
    return zc * lax.rsqrt(v + 1e-5) * g_ref[...] + be_ref[...]


def _t_layer_body(parts_ref, h_ref, dv_ref, b_ref, g_ref, be_ref, w_ref,
                  hn_ref, hpn_ref):
    z = _bn_block(parts_ref, h_ref, dv_ref, b_ref, g_ref, be_ref)
    hn = jnp.dot(z, w_ref[...], preferred_element_type=jnp.float32)
    _store_h_hp(hn, dv_ref[0:N, :], hn_ref, hpn_ref)


def _t_layer(parts, h_prev, dinv_col, b, g, be, W_next):
    return pl.pallas_call(_t_layer_body, out_shape=_H_OUT)(
        parts, h_prev, dinv_col, b, g, be, W_next)


def _t_final_body(parts_ref, h_ref, dv_ref, b_ref, g_ref, be_ref, batch_ref,
                  wm1_ref, bm1_ref, wm2_ref, bm2_ref, out_ref):
    z = _bn_block(parts_ref, h_ref, dv_ref, b_ref, g_ref, be_ref)
    bt = batch_ref[...]
    gid = lax.broadcasted_iota(jnp.int32, (NG, N), 0)
    oh = (gid == bt[None, :]).astype(jnp.float32)
    sums = jnp.dot(oh, z, preferred_element_type=jnp.float32)
    cnt = jnp.dot(oh, jnp.ones((N, 1), jnp.float32),
                  preferred_element_type=jnp.float32)
    pooled = sums / jnp.maximum(cnt, 1.0)
    h1 = jnp.maximum(
        jnp.dot(pooled, wm1_ref[...], preferred_element_type=jnp.float32)
        + bm1_ref[...], 0.0)
    out_ref[...] = (jnp.dot(h1, wm2_ref[...], preferred_element_type=jnp.float32)
                    + bm2_ref[...])


def _t_final(parts, h_prev, dinv_col, b, g, be, batch, Wm1, bm1, Wm2, bm2):
    return pl.pallas_call(
        _t_final_body,
        out_shape=jax.ShapeDtypeStruct((NG, NCOUT), jnp.float32),
    )(parts, h_prev, dinv_col, b, g, be, batch, Wm1, bm1, Wm2, bm2)


def _pack_edges(src, dst, edge_attr):
    parts = []
    for arr, padval in ((src, 0), (dst, NPAD - 1)):
        a = arr.reshape(NW, EPW)
        pad = jnp.full((NW, PADE), padval, jnp.int32)
        parts.append(jnp.concatenate([a, pad], axis=1).reshape(NW, NCH2, 1, KE))
    epk = jnp.concatenate(parts, axis=2).reshape(NW * NCH2, 2, KE)
    wpad = jnp.zeros((NW, PADE), jnp.float32)
    ew = jnp.concatenate([edge_attr.reshape(NW, EPW), wpad],
                         axis=1).reshape(NW * NCH2, KE)
    return epk, ew


def kernel(x, edge_index, edge_attr, batch,
           W0, b0, g0, be0, W1, b1, g1, be1, W2, b2, g2, be2,
           Wm1, bm1, Wm2, bm2):
    src = edge_index[0]
    dst = edge_index[1]
    epk, ew = _pack_edges(src, dst, edge_attr)
    deg_parts = _sc_deg(dst, edge_attr)
    dinv = _t_dinv(deg_parts)
    dinv_col = dinv[:, None]
    h0, hp0 = _t_mm0(x, W0, dinv_col)
    parts = _sc_edge(hp0, epk, ew)
    h1, hp1 = _t_layer(parts, h0, dinv_col, b0, g0, be0, W1)
    parts = _sc_edge(hp1, epk, ew)
    h2, hp2 = _t_layer(parts, h1, dinv_col, b1, g1, be1, W2)
    parts = _sc_edge(hp2, epk, ew)
    return _t_final(parts, h2, dinv_col, b2, g2, be2, batch, Wm1, bm1, Wm2, bm2)


# speedup vs baseline: 1.8914x; 1.8914x over previous
import functools

import jax
import jax.numpy as jnp
from jax import lax
from jax.experimental import pallas as pl
from jax.experimental.pallas import tpu as pltpu
from jax.experimental.pallas import tpu_sc as plsc

N = 10000
NPAD = 10240
E = 320000
D = 128
NG = 8
NCOUT = 2
NW = 32
EPW = E // NW
K = 80
NCHUNK = EPW // K
RPT = NPAD // 16

_mesh = plsc.VectorSubcoreMesh(core_axis_name="c", subcore_axis_name="s")

_GATHER_DNUMS = lax.GatherDimensionNumbers(
    offset_dims=(), collapsed_slice_dims=(0,), start_index_map=(0,))


def _lane_bcast(vec, l):
    idx = jnp.full((16, 1), l, jnp.int32)
    return lax.gather(vec, idx, _GATHER_DNUMS, (1,),
                      mode=lax.GatherScatterMode.PROMISE_IN_BOUNDS)


@functools.partial(
    pl.kernel,
    out_type=jax.ShapeDtypeStruct((2, NPAD), jnp.float32),
    mesh=_mesh,
    scratch_types=[
        pltpu.VMEM((K,), jnp.int32),
        pltpu.VMEM((K,), jnp.float32),
        pltpu.VMEM((RPT,), jnp.float32),
        pltpu.VMEM_SHARED((NPAD,), jnp.float32),
    ],
)
def _sc_deg(dst_hbm, w_hbm, out_hbm, dst_v, w_v, zb_v, acc_sh):
    c = lax.axis_index("c")
    s = lax.axis_index("s")
    wid = s * 2 + c
    for i in range(RPT // 16):
        zb_v[pl.ds(i * 16, 16)] = jnp.zeros((16,), jnp.float32)
    pltpu.sync_copy(zb_v, acc_sh.at[pl.ds(s * RPT, RPT)])
    plsc.subcore_barrier()

    def chunk(ci, carry):
        off = pl.multiple_of(wid * EPW + ci * K, 8)
        pltpu.sync_copy(dst_hbm.at[pl.ds(off, K)], dst_v)
        pltpu.sync_copy(w_hbm.at[pl.ds(off, K)], w_v)

        def absgrp(g, cc):
            w_v[pl.ds(g * 16, 16)] = jnp.abs(w_v[pl.ds(g * 16, 16)])
            return cc

        lax.fori_loop(0, K // 16, absgrp, 0)
        pltpu.sync_copy(w_v, acc_sh.at[dst_v], add=True)
        return carry

    lax.fori_loop(0, NCHUNK, chunk, 0)
    plsc.subcore_barrier()
    pltpu.sync_copy(acc_sh.at[pl.ds(s * RPT, RPT)],
                    out_hbm.at[c, pl.ds(s * RPT, RPT)])


KE = 112
EPWP = 10080
NCH2 = EPWP // KE
PADE = EPWP - EPW
NB = 3


def _scale_rows(wbufs, b, rbuf):
    def grp(g, cc):
        wvec = jnp.abs(wbufs[b, pl.ds(g * 16, 16)])
        base = g * 16
        for l in range(16):
            sv = _lane_bcast(wvec, l)
            e = base + l
            for j in range(D // 16):
                rbuf[e, pl.ds(j * 16, 16)] = rbuf[e, pl.ds(j * 16, 16)] * sv
        return cc

    lax.fori_loop(0, KE // 16, grp, 0)


@functools.partial(
    pl.kernel,
    out_type=jax.ShapeDtypeStruct((2, NPAD, D), jnp.float32),
    mesh=_mesh,
    scratch_types=[
        pltpu.VMEM((NB, 2, KE), jnp.int32),
        pltpu.VMEM((NB, KE), jnp.float32),
        pltpu.VMEM((KE,), jnp.int32),
        pltpu.VMEM((KE,), jnp.int32),
        pltpu.VMEM((KE,), jnp.int32),
        pltpu.VMEM((KE,), jnp.int32),
        pltpu.VMEM((KE,), jnp.int32),
        pltpu.VMEM((KE,), jnp.int32),
        pltpu.VMEM((KE, D), jnp.float32),
        pltpu.VMEM((KE, D), jnp.float32),
        pltpu.VMEM((KE, D), jnp.float32),
        pltpu.VMEM_SHARED((NPAD, D), jnp.float32),
        pltpu.SemaphoreType.DMA,
        pltpu.SemaphoreType.DMA,
        pltpu.SemaphoreType.DMA,
        pltpu.SemaphoreType.DMA,
        pltpu.SemaphoreType.DMA,
        pltpu.SemaphoreType.DMA,
        pltpu.SemaphoreType.DMA,
        pltpu.SemaphoreType.DMA,
        pltpu.SemaphoreType.DMA,
    ],
)
def _sc_edge(hp_hbm, epk_hbm, ew_hbm, out_hbm,
             ebuf, wbufs, srcb0, srcb1, srcb2, dstb0, dstb1, dstb2,
             rbuf0, rbuf1, rbuf2, acc_sh,
             gsem0, gsem1, gsem2, ssem0, ssem1, ssem2, esem0, esem1, esem2):
    c = lax.axis_index("c")
    s = lax.axis_index("s")
    wid = s * 2 + c
    srcb = (srcb0, srcb1, srcb2)
    dstb = (dstb0, dstb1, dstb2)
    rbuf = (rbuf0, rbuf1, rbuf2)
    gsem = (gsem0, gsem1, gsem2)
    ssem = (ssem0, ssem1, ssem2)
    esem = (esem0, esem1, esem2)
    row0 = wid * NCH2

    pltpu.sync_copy(hp_hbm.at[pl.ds(s * RPT, RPT)], acc_sh.at[pl.ds(s * RPT, RPT)])

    def zrow(r, cc):
        for j in range(D // 16):
            rbuf1[r, pl.ds(j * 16, 16)] = jnp.zeros((16,), jnp.float32)
            rbuf2[r, pl.ds(j * 16, 16)] = jnp.zeros((16,), jnp.float32)
        return cc

    lax.fori_loop(0, KE, zrow, 0)
    for g in range(KE // 16):
        dstb1[pl.ds(g * 16, 16)] = jnp.zeros((16,), jnp.int32)
        dstb2[pl.ds(g * 16, 16)] = jnp.zeros((16,), jnp.int32)
    plsc.subcore_barrier()
    pltpu.async_copy(rbuf1, acc_sh.at[dstb1], ssem1, add=True)
    pltpu.async_copy(rbuf2, acc_sh.at[dstb2], ssem2, add=True)

    def _edata_async(j, x):
        pltpu.async_copy(epk_hbm.at[row0 + j], ebuf.at[x], esem[x])
        pltpu.async_copy(ew_hbm.at[row0 + j], wbufs.at[x], esem[x])

    def _edata_drain(x):
        pltpu.make_async_copy(epk_hbm.at[row0], ebuf.at[x], esem[x]).wait()
        pltpu.make_async_copy(ew_hbm.at[row0], wbufs.at[x], esem[x]).wait()

    def _dcp(x):
        def body(g, cc):
            srcb[x][pl.ds(g * 16, 16)] = ebuf[x, 0, pl.ds(g * 16, 16)]
            dstb[x][pl.ds(g * 16, 16)] = ebuf[x, 1, pl.ds(g * 16, 16)]
            return cc

        lax.fori_loop(0, KE // 16, body, 0)

    _edata_async(0, 0)
    _edata_async(1, 1)
    _edata_drain(0)
    _dcp(0)
    pltpu.async_copy(hp_hbm.at[srcb0], rbuf0, gsem0)

    def triple(t, carry):
        for b in (0, 1, 2):
            i = 3 * t + b
            o = (b + 1) % NB
            n = (b + 2) % NB
            j = i + 1

            @pl.when(j < NCH2)
            def _():
                @pl.when(j + 1 < NCH2)
                def _():
                    _edata_async(j + 1, n)

                pltpu.make_async_copy(hp_hbm.at[pl.ds(0, KE)], rbuf[o],
                                      ssem[o]).wait()
                _edata_drain(o)
                _dcp(o)
                pltpu.async_copy(hp_hbm.at[srcb[o]], rbuf[o], gsem[o])

            pltpu.make_async_copy(hp_hbm.at[pl.ds(0, KE)], rbuf[b],
                                  gsem[b]).wait()
            _scale_rows(wbufs, b, rbuf[b])
            pltpu.async_copy(rbuf[b], acc_sh.at[dstb[b]], ssem[b], add=True)
        return carry

    lax.fori_loop(0, NCH2 // 3, triple, 0)
    for b in (0, 1, 2):
        pltpu.make_async_copy(hp_hbm.at[pl.ds(0, KE)], rbuf[b], ssem[b]).wait()
    plsc.subcore_barrier()
    pltpu.sync_copy(acc_sh.at[pl.ds(s * RPT, RPT)],
                    out_hbm.at[c, pl.ds(s * RPT, RPT)])


def _t_dinv_body(dp_ref, out_ref):
    dp = dp_ref[...]
    out_ref[...] = lax.rsqrt(1.0 + dp[0] + dp[1])


def _t_dinv(deg_parts):
    return pl.pallas_call(
        _t_dinv_body,
        out_shape=jax.ShapeDtypeStruct((NPAD,), jnp.float32),
    )(deg_parts)


def _store_h_hp(h, dv, h_ref, hp_ref):
    zpad = jnp.zeros((NPAD - N, D), jnp.float32)
    h_ref[0:N, :] = h
    h_ref[N:NPAD, :] = zpad
    hp_ref[0:N, :] = dv * h
    hp_ref[N:NPAD, :] = zpad


def _t_mm0_body(x_ref, w_ref, dv_ref, h_ref, hp_ref):
    h = jnp.dot(x_ref[...], w_ref[...], preferred_element_type=jnp.float32)
    _store_h_hp(h, dv_ref[0:N, :], h_ref, hp_ref)


_H_OUT = [jax.ShapeDtypeStruct((NPAD, D), jnp.float32),
          jax.ShapeDtypeStruct((NPAD, D), jnp.float32)]


def _t_mm0(x, W, dinv_col):
    return pl.pallas_call(_t_mm0_body, out_shape=_H_OUT)(x, W, dinv_col)


def _bn_block(parts_ref, h_ref, dv_ref, b_ref, g_ref, be_ref):
    p = parts_ref[0, 0:N, :] + parts_ref[1, 0:N, :]
    dv = dv_ref[0:N, :]
    h = h_ref[0:N, :]
    hp = dv * h
    z = dv * (p - hp) + b_ref[...]
    z = jnp.where(z > 0, z, 0.2 * z)
    m = jnp.sum(z, axis=0) * (1.0 / N)
    zc = z - m
    v = jnp.sum(zc * zc, axis=0) * (1.0 / N)
    return zc * lax.rsqrt(v + 1e-5) * g_ref[...] + be_ref[...]


def _t_layer_body(parts_ref, h_ref, dv_ref, b_ref, g_ref, be_ref, w_ref,
                  hn_ref, hpn_ref):
    z = _bn_block(parts_ref, h_ref, dv_ref, b_ref, g_ref, be_ref)
    hn = jnp.dot(z, w_ref[...], preferred_element_type=jnp.float32)
    _store_h_hp(hn, dv_ref[0:N, :], hn_ref, hpn_ref)


def _t_layer(parts, h_prev, dinv_col, b, g, be, W_next):
    return pl.pallas_call(_t_layer_body, out_shape=_H_OUT)(
        parts, h_prev, dinv_col, b, g, be, W_next)


def _t_final_body(parts_ref, h_ref, dv_ref, b_ref, g_ref, be_ref, batch_ref,
                  wm1_ref, bm1_ref, wm2_ref, bm2_ref, out_ref):
    z = _bn_block(parts_ref, h_ref, dv_ref, b_ref, g_ref, be_ref)
    bt = batch_ref[...]
    gid = lax.broadcasted_iota(jnp.int32, (NG, N), 0)
    oh = (gid == bt[None, :]).astype(jnp.float32)
    sums = jnp.dot(oh, z, preferred_element_type=jnp.float32)
    cnt = jnp.dot(oh, jnp.ones((N, 1), jnp.float32),
                  preferred_element_type=jnp.float32)
    pooled = sums / jnp.maximum(cnt, 1.0)
    h1 = jnp.maximum(
        jnp.dot(pooled, wm1_ref[...], preferred_element_type=jnp.float32)
        + bm1_ref[...], 0.0)
    out_ref[...] = (jnp.dot(h1, wm2_ref[...], preferred_element_type=jnp.float32)
                    + bm2_ref[...])


def _t_final(parts, h_prev, dinv_col, b, g, be, batch, Wm1, bm1, Wm2, bm2):
    return pl.pallas_call(
        _t_final_body,
        out_shape=jax.ShapeDtypeStruct((NG, NCOUT), jnp.float32),
    )(parts, h_prev, dinv_col, b, g, be, batch, Wm1, bm1, Wm2, bm2)


def _pack_edges(src, dst, edge_attr):
    parts = []
    for arr, padval in ((src, 0), (dst, NPAD - 1)):
        a = arr.reshape(NW, EPW)
        pad = jnp.full((NW, PADE), padval, jnp.int32)
        parts.append(jnp.concatenate([a, pad], axis=1).reshape(NW, NCH2, 1, KE))
    epk = jnp.concatenate(parts, axis=2).reshape(NW * NCH2, 2, KE)
    wpad = jnp.zeros((NW, PADE), jnp.float32)
    ew = jnp.concatenate([edge_attr.reshape(NW, EPW), wpad],
                         axis=1).reshape(NW * NCH2, KE)
    return epk, ew


def kernel(x, edge_index, edge_attr, batch,
           W0, b0, g0, be0, W1, b1, g1, be1, W2, b2, g2, be2,
           Wm1, bm1, Wm2, bm2):
    src = edge_index[0]
    dst = edge_index[1]
    epk, ew = _pack_edges(src, dst, edge_attr)
    deg_parts = _sc_deg(dst, edge_attr)
    dinv = _t_dinv(deg_parts)
    dinv_col = dinv[:, None]
    h0, hp0 = _t_mm0(x, W0, dinv_col)
    parts = _sc_edge(hp0, epk, ew)
    h1, hp1 = _t_layer(parts, h0, dinv_col, b0, g0, be0, W1)
    parts = _sc_edge(hp1, epk, ew)
    h2, hp2 = _t_layer(parts, h1, dinv_col, b1, g1, be1, W2)
    parts = _sc_edge(hp2, epk, ew)
    return _t_final(parts, h2, dinv_col, b2, g2, be2, batch, Wm1, bm1, Wm2, bm2)
